# baseline (speedup 1.0000x reference)
import functools
import math

import numpy as np
import jax
import jax.numpy as jnp
from jax.experimental import pallas as pl
from jax.experimental.pallas import tpu as pltpu
from jax.sharding import Mesh, PartitionSpec as P

KEEP_RATIO = 0.5
MIN_TOKENS = 1


def _score_body(aw_ref, scores_ref, acc_ref):
    h = pl.program_id(1)
    acc_ref[h, :] = jnp.sum(aw_ref[0, 0], axis=0)

    @pl.when(h == pl.num_programs(1) - 1)
    def _():
        scores_ref[0, 0, :] = jnp.mean(acc_ref[...], axis=0)


def _prune_body(k, scores_ref, hs_ref, out_ref, mask_ref):
    s = scores_ref[0, 0, :]
    T = s.shape[0]
    s_i = s[:, None]
    s_j = s[None, :]
    i_idx = jax.lax.broadcasted_iota(jnp.int32, (T, T), 0)
    j_idx = jax.lax.broadcasted_iota(jnp.int32, (T, T), 1)
    beats = (s_j > s_i) | ((s_j == s_i) & (j_idx < i_idx))
    rank = jnp.sum(beats.astype(jnp.int32), axis=1)
    pos = jax.lax.broadcasted_iota(jnp.int32, (T,), 0)
    keep = (rank < k) | (pos == 0)
    mask_ref[0, 0, :] = keep.astype(jnp.int32)
    out_ref[0] = hs_ref[0] * keep.astype(out_ref.dtype)[:, None]


def _local_prune(k, hs, aw):
    Bl, T, D = hs.shape
    _, H, _, _ = aw.shape

    scores = pl.pallas_call(
        _score_body,
        grid=(Bl, H),
        in_specs=[pl.BlockSpec((1, 1, T, T), lambda b, h: (b, h, 0, 0))],
        out_specs=pl.BlockSpec((1, 1, T), lambda b, h: (b, 0, 0)),
        out_shape=jax.ShapeDtypeStruct((Bl, 1, T), jnp.float32),
        scratch_shapes=[pltpu.VMEM((H, T), jnp.float32)],
        compiler_params=pltpu.CompilerParams(
            dimension_semantics=("arbitrary", "arbitrary"),
        ),
    )(aw)

    pruned, mask_i32 = pl.pallas_call(
        functools.partial(_prune_body, k),
        grid=(Bl,),
        in_specs=[
            pl.BlockSpec((1, 1, T), lambda b: (b, 0, 0)),
            pl.BlockSpec((1, T, D), lambda b: (b, 0, 0)),
        ],
        out_specs=[
            pl.BlockSpec((1, T, D), lambda b: (b, 0, 0)),
            pl.BlockSpec((1, 1, T), lambda b: (b, 0, 0)),
        ],
        out_shape=[
            jax.ShapeDtypeStruct((Bl, T, D), hs.dtype),
            jax.ShapeDtypeStruct((Bl, 1, T), jnp.int32),
        ],
    )(scores, hs)
    return pruned, mask_i32


@jax.jit
def kernel(hidden_states, attention_weights):
    B, T, D = hidden_states.shape
    k = min(max(MIN_TOKENS, math.ceil(KEEP_RATIO * T)), T)
    fn = functools.partial(_local_prune, k)

    devs = jax.devices()
    n_shard = max(d for d in range(1, min(len(devs), B) + 1) if B % d == 0)
    if n_shard > 1:
        mesh = Mesh(np.asarray(devs[:n_shard]), ("x",))
        fn = jax.shard_map(
            fn, mesh=mesh,
            in_specs=(P("x"), P("x")),
            out_specs=(P("x"), P("x")),
            check_vma=False,
        )
    pruned, mask_i32 = fn(hidden_states, attention_weights)
    return (pruned, mask_i32.reshape(B, T).astype(bool))


# speedup vs baseline: 4.5386x; 4.5386x over previous
import functools
import math

import jax
import jax.numpy as jnp
from jax.experimental import pallas as pl
from jax.experimental.pallas import tpu as pltpu

KEEP_RATIO = 0.5
MIN_TOKENS = 1


def _fused_body(k, H, aw_ref, hs_ref, out_ref, mask_ref, acc_ref, keep_ref):
    h = pl.program_id(1)

    @pl.when(h < H)
    def _():
        acc_ref[h, :] = jnp.sum(aw_ref[0, 0], axis=0)

    @pl.when(h == H)
    def _():
        s = jnp.mean(acc_ref[...], axis=0)
        T = s.shape[0]
        s_i = s[:, None]
        s_j = s[None, :]
        i_idx = jax.lax.broadcasted_iota(jnp.int32, (T, T), 0)
        j_idx = jax.lax.broadcasted_iota(jnp.int32, (T, T), 1)
        beats = (s_j > s_i) | ((s_j == s_i) & (j_idx < i_idx))
        rank = jnp.sum(beats.astype(jnp.int32), axis=1)
        pos = jax.lax.broadcasted_iota(jnp.int32, (T,), 0)
        keep = (rank < k) | (pos == 0)
        keepf = keep.astype(jnp.float32)
        keep_ref[0, :] = keepf
        mask_ref[0, 0, :] = keep.astype(jnp.int32)
        out_ref[0] = hs_ref[0] * keepf[:, None]

    @pl.when(h == H + 1)
    def _():
        out_ref[0] = hs_ref[0] * keep_ref[0, :][:, None]


@jax.jit
def kernel(hidden_states, attention_weights):
    B, T, D = hidden_states.shape
    _, H, _, _ = attention_weights.shape
    k = min(max(MIN_TOKENS, math.ceil(KEEP_RATIO * T)), T)
    DS = 2 if D % 2 == 0 else 1
    Dc = D // DS

    pruned, mask_i32 = pl.pallas_call(
        functools.partial(_fused_body, k, H),
        grid=(B, H + DS),
        in_specs=[
            pl.BlockSpec(
                (1, 1, T, T), lambda b, h: (b, jnp.minimum(h, H - 1), 0, 0)
            ),
            pl.BlockSpec(
                (1, T, Dc),
                lambda b, h: (b, 0, jnp.clip(h - H, 0, DS - 1)),
            ),
        ],
        out_specs=[
            pl.BlockSpec(
                (1, T, Dc),
                lambda b, h: (b, 0, jnp.clip(h - H, 0, DS - 1)),
            ),
            pl.BlockSpec((1, 1, T), lambda b, h: (b, 0, 0)),
        ],
        out_shape=[
            jax.ShapeDtypeStruct((B, T, D), hidden_states.dtype),
            jax.ShapeDtypeStruct((B, 1, T), jnp.int32),
        ],
        scratch_shapes=[
            pltpu.VMEM((H, T), jnp.float32),
            pltpu.VMEM((8, T), jnp.float32),
        ],
        compiler_params=pltpu.CompilerParams(
            dimension_semantics=("arbitrary", "arbitrary"),
        ),
    )(attention_weights, hidden_states)

    return (pruned, mask_i32.reshape(B, T).astype(bool))
